# baseline (speedup 1.0000x reference)
import functools

import jax
import jax.numpy as jnp
from jax import lax
from jax.experimental import pallas as pl
from jax.experimental.pallas import tpu as pltpu
from jax.experimental.pallas import tpu_sc as plsc

VOCAB = 1000000
DIM = 64
BATCH = 16384
CTX_LEN = 20
NUM_NEG = 5

NC = 2
NS = 16
NW = NC * NS
LANES = 16
DV = DIM // LANES

B_PER_W = BATCH // NW
IDX_PER_CHUNK = 320
CHUNK_A = IDX_PER_CHUNK // CTX_LEN
N_CHUNK_A = B_PER_W // CHUNK_A
CHUNK_B = IDX_PER_CHUNK // NUM_NEG
N_CHUNK_B = B_PER_W // CHUNK_B

_GDN = lax.GatherDimensionNumbers(
    offset_dims=(), collapsed_slice_dims=(0,), start_index_map=(0,))


def _lane_shuffle(x, idx):
  return lax.gather(x, idx[:, None], _GDN, (1,),
                    mode=lax.GatherScatterMode.PROMISE_IN_BOUNDS)


def _sc_body(ctx_idx_hbm, neg_idx_hbm, embed_hbm, context_hbm, out_hbm,
             ctx2d_v, neg2d_v, ctx_flat, neg_flat, rows_0, rows_1,
             mean_v, out_v, sem_0, sem_1):
  wid = lax.axis_index("s") * NC + lax.axis_index("c")
  b0 = wid * B_PER_W
  lane = lax.iota(jnp.int32, LANES)
  shuf_idx = [lane ^ s for s in (8, 4, 2, 1)]
  k_masks = [lane == k for k in range(NUM_NEG)]
  neg_mask = lane < NUM_NEG

  pltpu.sync_copy(ctx_idx_hbm.at[pl.ds(b0, B_PER_W), :], ctx2d_v)
  pltpu.sync_copy(neg_idx_hbm.at[pl.ds(b0, B_PER_W), :], neg2d_v)

  def flatten_body(b, _):
    base_c = b * CTX_LEN
    v0 = ctx2d_v[b, pl.ds(0, LANES)]
    v1 = ctx2d_v[b, pl.ds(CTX_LEN - LANES, LANES)]
    plsc.store_scatter(ctx_flat, [base_c + lane], v0)
    plsc.store_scatter(ctx_flat, [base_c + (CTX_LEN - LANES) + lane], v1)
    vn = plsc.load_gather(neg2d_v, [jnp.full((LANES,), b, jnp.int32), lane],
                          mask=neg_mask)
    plsc.store_scatter(neg_flat, [b * NUM_NEG + lane], vn, mask=neg_mask)
    return _

  lax.fori_loop(0, B_PER_W, flatten_body, 0, unroll=False)

  rows_bufs = (rows_0, rows_1)
  sems = (sem_0, sem_1)

  def fire_gather(table_hbm, idx_ref, chunk, buf):
    return pltpu.async_copy(
        table_hbm.at[idx_ref.at[pl.ds(chunk * IDX_PER_CHUNK,
                                      IDX_PER_CHUNK)]],
        rows_bufs[buf], sems[buf])

  pending = {0: fire_gather(embed_hbm, ctx_flat, 0, 0)}
  for g in range(N_CHUNK_A):
    cur = g % 2
    if g + 1 < N_CHUNK_A:
      pending[1 - cur] = fire_gather(embed_hbm, ctx_flat, g + 1, 1 - cur)
    pending.pop(cur).wait()
    rows = rows_bufs[cur]

    def mean_body(i, _, g=g, rows=rows):
      r = i * CTX_LEN
      accs = [rows[r, pl.ds(q * LANES, LANES)] for q in range(DV)]
      for j in range(1, CTX_LEN):
        for q in range(DV):
          accs[q] = accs[q] + rows[r + j, pl.ds(q * LANES, LANES)]
      b = g * CHUNK_A + i
      for q in range(DV):
        mean_v[b, pl.ds(q * LANES, LANES)] = accs[q] * (1.0 / CTX_LEN)
      return _

    lax.fori_loop(0, CHUNK_A, mean_body, 0, unroll=False)

  pending = {0: fire_gather(context_hbm, neg_flat, 0, 0)}
  for h in range(N_CHUNK_B):
    cur = h % 2
    if h + 1 < N_CHUNK_B:
      pending[1 - cur] = fire_gather(context_hbm, neg_flat, h + 1, 1 - cur)
    pending.pop(cur).wait()
    rows = rows_bufs[cur]

    def dot_body(i, _, h=h, rows=rows):
      b = h * CHUNK_B + i
      m = [mean_v[b, pl.ds(q * LANES, LANES)] for q in range(DV)]
      acc = jnp.zeros((LANES,), jnp.float32)
      for k in range(NUM_NEG):
        r = i * NUM_NEG + k
        p = rows[r, pl.ds(0, LANES)] * m[0]
        for q in range(1, DV):
          p = p + rows[r, pl.ds(q * LANES, LANES)] * m[q]
        for sidx in shuf_idx:
          p = p + _lane_shuffle(p, sidx)
        acc = jnp.where(k_masks[k], p, acc)
      plsc.store_scatter(out_v, [jnp.full((LANES,), i, jnp.int32), lane],
                         acc, mask=neg_mask)
      return _

    lax.fori_loop(0, CHUNK_B, dot_body, 0, unroll=False)
    pltpu.sync_copy(out_v,
                    out_hbm.at[pl.ds(b0 + h * CHUNK_B, CHUNK_B), :])


@jax.jit
def _cbow_scores(ctx_idx, neg_idx, W_embed, W_context):
  mesh = plsc.VectorSubcoreMesh(
      core_axis_name="c", subcore_axis_name="s",
      num_cores=NC, num_subcores=NS)
  fn = pl.kernel(
      _sc_body,
      out_type=jax.ShapeDtypeStruct((BATCH, NUM_NEG), jnp.float32),
      mesh=mesh,
      compiler_params=pltpu.CompilerParams(use_tc_tiling_on_sc=False,
                                           needs_layout_passes=False),
      scratch_types=[
          pltpu.VMEM((B_PER_W, CTX_LEN), jnp.int32),
          pltpu.VMEM((B_PER_W, NUM_NEG), jnp.int32),
          pltpu.VMEM((B_PER_W * CTX_LEN,), jnp.int32),
          pltpu.VMEM((B_PER_W * NUM_NEG,), jnp.int32),
          pltpu.VMEM((IDX_PER_CHUNK, DIM), jnp.float32),
          pltpu.VMEM((IDX_PER_CHUNK, DIM), jnp.float32),
          pltpu.VMEM((B_PER_W, DIM), jnp.float32),
          pltpu.VMEM((CHUNK_B, NUM_NEG), jnp.float32),
          pltpu.SemaphoreType.DMA,
          pltpu.SemaphoreType.DMA,
      ],
  )
  return fn(ctx_idx, neg_idx, W_embed, W_context)


def kernel(context_words, negative_words, W_embed, W_context):
  return _cbow_scores(context_words, negative_words, W_embed, W_context)


# speedup vs baseline: 1.1506x; 1.1506x over previous
import functools

import jax
import jax.numpy as jnp
from jax import lax
from jax.experimental import pallas as pl
from jax.experimental.pallas import tpu as pltpu
from jax.experimental.pallas import tpu_sc as plsc

VOCAB = 1000000
DIM = 64
BATCH = 16384
CTX_LEN = 20
NUM_NEG = 5

NC = 2
NS = 16
NW = NC * NS
LANES = 16
DV = DIM // LANES

B_PER_W = BATCH // NW
IDX_PER_CHUNK = 160
CHUNK_A = IDX_PER_CHUNK // CTX_LEN
N_CHUNK_A = B_PER_W // CHUNK_A
CHUNK_B = IDX_PER_CHUNK // NUM_NEG
N_CHUNK_B = B_PER_W // CHUNK_B
CAT = 2 * DIM

_GDN = lax.GatherDimensionNumbers(
    offset_dims=(), collapsed_slice_dims=(0,), start_index_map=(0,))


def _lane_shuffle(x, idx):
  return lax.gather(x, idx[:, None], _GDN, (1,),
                    mode=lax.GatherScatterMode.PROMISE_IN_BOUNDS)


def _sc_body(ctx_idx_hbm, neg_idx_hbm, wcat_hbm, out_hbm,
             ctx2d_v, neg2d_v, ctx_flat, neg_flat, rows_0, rows_1,
             mean_v, out_v, sem_0, sem_1):
  wid = lax.axis_index("s") * NC + lax.axis_index("c")
  b0 = wid * B_PER_W
  lane = lax.iota(jnp.int32, LANES)
  shuf_idx = [lane ^ s for s in (8, 4, 2, 1)]
  k_masks = [lane == k for k in range(NUM_NEG)]
  neg_mask = lane < NUM_NEG

  pltpu.sync_copy(ctx_idx_hbm.at[pl.ds(b0, B_PER_W), :], ctx2d_v)
  pltpu.sync_copy(neg_idx_hbm.at[pl.ds(b0, B_PER_W), :], neg2d_v)

  def flatten_body(b, _):
    base_c = b * CTX_LEN
    v0 = ctx2d_v[b, pl.ds(0, LANES)]
    v1 = ctx2d_v[b, pl.ds(CTX_LEN - LANES, LANES)]
    plsc.store_scatter(ctx_flat, [base_c + lane], v0)
    plsc.store_scatter(ctx_flat, [base_c + (CTX_LEN - LANES) + lane], v1)
    vn = plsc.load_gather(neg2d_v, [jnp.full((LANES,), b, jnp.int32), lane],
                          mask=neg_mask)
    plsc.store_scatter(neg_flat, [b * NUM_NEG + lane], vn, mask=neg_mask)
    return _

  lax.fori_loop(0, B_PER_W, flatten_body, 0, unroll=False)

  rows_bufs = (rows_0, rows_1)
  sems = (sem_0, sem_1)

  def fire_gather(idx_ref, chunk, buf):
    pltpu.async_copy(
        wcat_hbm.at[idx_ref.at[pl.ds(chunk * IDX_PER_CHUNK,
                                     IDX_PER_CHUNK)]],
        rows_bufs[buf], sems[buf])

  def wait_gather(buf):
    pltpu.make_async_copy(
        wcat_hbm.at[pl.ds(0, IDX_PER_CHUNK), :],
        rows_bufs[buf], sems[buf]).wait()

  def compute_mean(c, buf):
    rows = rows_bufs[buf]

    def mean_body(i, _):
      r = i * CTX_LEN
      accs = [rows[r, pl.ds(q * LANES, LANES)] for q in range(DV)]
      for j in range(1, CTX_LEN):
        for q in range(DV):
          accs[q] = accs[q] + rows[r + j, pl.ds(q * LANES, LANES)]
      b = c * CHUNK_A + i
      for q in range(DV):
        mean_v[b, pl.ds(q * LANES, LANES)] = accs[q] * (1.0 / CTX_LEN)
      return _

    lax.fori_loop(0, CHUNK_A, mean_body, 0, unroll=False)

  fire_gather(ctx_flat, 0, 0)

  def phase_a_pair(gp, _):
    c0 = 2 * gp
    fire_gather(ctx_flat, jnp.minimum(c0 + 1, N_CHUNK_A - 1), 1)
    wait_gather(0)
    compute_mean(c0, 0)
    fire_gather(ctx_flat, jnp.minimum(c0 + 2, N_CHUNK_A - 1), 0)
    wait_gather(1)
    compute_mean(c0 + 1, 1)
    return _

  lax.fori_loop(0, N_CHUNK_A // 2, phase_a_pair, 0, unroll=False)
  wait_gather(0)

  def compute_dots(c, buf):
    rows = rows_bufs[buf]

    def dot_body(i, _):
      b = c * CHUNK_B + i
      m = [mean_v[b, pl.ds(q * LANES, LANES)] for q in range(DV)]
      acc = jnp.zeros((LANES,), jnp.float32)
      for k in range(NUM_NEG):
        r = i * NUM_NEG + k
        p = rows[r, pl.ds(DIM, LANES)] * m[0]
        for q in range(1, DV):
          p = p + rows[r, pl.ds(DIM + q * LANES, LANES)] * m[q]
        for sidx in shuf_idx:
          p = p + _lane_shuffle(p, sidx)
        acc = jnp.where(k_masks[k], p, acc)
      plsc.store_scatter(out_v, [jnp.full((LANES,), i, jnp.int32), lane],
                         acc, mask=neg_mask)
      return _

    lax.fori_loop(0, CHUNK_B, dot_body, 0, unroll=False)
    pltpu.sync_copy(out_v,
                    out_hbm.at[pl.ds(b0 + c * CHUNK_B, CHUNK_B), :])

  fire_gather(neg_flat, 0, 0)

  def phase_b_pair(hp, _):
    c0 = 2 * hp
    fire_gather(neg_flat, jnp.minimum(c0 + 1, N_CHUNK_B - 1), 1)
    wait_gather(0)
    compute_dots(c0, 0)
    fire_gather(neg_flat, jnp.minimum(c0 + 2, N_CHUNK_B - 1), 0)
    wait_gather(1)
    compute_dots(c0 + 1, 1)
    return _

  lax.fori_loop(0, N_CHUNK_B // 2, phase_b_pair, 0, unroll=False)
  wait_gather(0)


@jax.jit
def _cbow_scores(ctx_idx, neg_idx, w_cat):
  mesh = plsc.VectorSubcoreMesh(
      core_axis_name="c", subcore_axis_name="s",
      num_cores=NC, num_subcores=NS)
  fn = pl.kernel(
      _sc_body,
      out_type=jax.ShapeDtypeStruct((BATCH, NUM_NEG), jnp.float32),
      mesh=mesh,
      compiler_params=pltpu.CompilerParams(use_tc_tiling_on_sc=False,
                                           needs_layout_passes=False),
      scratch_types=[
          pltpu.VMEM((B_PER_W, CTX_LEN), jnp.int32),
          pltpu.VMEM((B_PER_W, NUM_NEG), jnp.int32),
          pltpu.VMEM((B_PER_W * CTX_LEN,), jnp.int32),
          pltpu.VMEM((B_PER_W * NUM_NEG,), jnp.int32),
          pltpu.VMEM((IDX_PER_CHUNK, CAT), jnp.float32),
          pltpu.VMEM((IDX_PER_CHUNK, CAT), jnp.float32),
          pltpu.VMEM((B_PER_W, DIM), jnp.float32),
          pltpu.VMEM((CHUNK_B, NUM_NEG), jnp.float32),
          pltpu.SemaphoreType.DMA,
          pltpu.SemaphoreType.DMA,
      ],
  )
  return fn(ctx_idx, neg_idx, w_cat)


def kernel(context_words, negative_words, W_embed, W_context):
  w_cat = jnp.concatenate([W_embed, W_context], axis=1)
  return _cbow_scores(context_words, negative_words, w_cat)


# speedup vs baseline: 1.5158x; 1.3175x over previous
import functools

import jax
import jax.numpy as jnp
from jax import lax
from jax.experimental import pallas as pl
from jax.experimental.pallas import tpu as pltpu
from jax.experimental.pallas import tpu_sc as plsc

VOCAB = 1000000
DIM = 64
BATCH = 16384
CTX_LEN = 20
NUM_NEG = 5

NC = 2
NS = 16
NW = NC * NS
LANES = 16
DV = DIM // LANES

B_PER_W = BATCH // NW
IDX_PER_CHUNK = 320
CHUNK_A = IDX_PER_CHUNK // CTX_LEN
N_CHUNK_A = B_PER_W // CHUNK_A
CHUNK_B = IDX_PER_CHUNK // NUM_NEG
N_CHUNK_B = B_PER_W // CHUNK_B

_GDN = lax.GatherDimensionNumbers(
    offset_dims=(), collapsed_slice_dims=(0,), start_index_map=(0,))


def _lane_shuffle(x, idx):
  return lax.gather(x, idx[:, None], _GDN, (1,),
                    mode=lax.GatherScatterMode.PROMISE_IN_BOUNDS)


def _sc_body(ctx_idx_hbm, neg_idx_hbm, wcat_hbm, out_hbm,
             ctx2d_v, neg2d_v, ctx_flat, neg_flat, rows_0, rows_1,
             mean_v, out_v, sem_0, sem_1):
  wid = lax.axis_index("s") * NC + lax.axis_index("c")
  b0 = wid * B_PER_W
  lane = lax.iota(jnp.int32, LANES)
  shuf_idx = [lane ^ s for s in (8, 4, 2, 1)]
  k_masks = [lane == k for k in range(NUM_NEG)]
  neg_mask = lane < NUM_NEG

  pltpu.sync_copy(ctx_idx_hbm.at[pl.ds(b0, B_PER_W), :], ctx2d_v)
  pltpu.sync_copy(neg_idx_hbm.at[pl.ds(b0, B_PER_W), :], neg2d_v)

  def flatten_body(b, _):
    base_c = b * CTX_LEN
    v0 = ctx2d_v[b, pl.ds(0, LANES)] * 2
    v1 = ctx2d_v[b, pl.ds(CTX_LEN - LANES, LANES)] * 2
    plsc.store_scatter(ctx_flat, [base_c + lane], v0)
    plsc.store_scatter(ctx_flat, [base_c + (CTX_LEN - LANES) + lane], v1)
    vn = plsc.load_gather(neg2d_v, [jnp.full((LANES,), b, jnp.int32), lane],
                          mask=neg_mask)
    plsc.store_scatter(neg_flat, [b * NUM_NEG + lane], vn * 2 + 1,
                       mask=neg_mask)
    return _

  lax.fori_loop(0, B_PER_W, flatten_body, 0, unroll=False)

  rows_bufs = (rows_0, rows_1)
  sems = (sem_0, sem_1)

  def fire_gather(idx_ref, chunk, buf):
    pltpu.async_copy(
        wcat_hbm.at[idx_ref.at[pl.ds(chunk * IDX_PER_CHUNK,
                                     IDX_PER_CHUNK)]],
        rows_bufs[buf], sems[buf])

  def wait_gather(buf):
    pltpu.make_async_copy(
        wcat_hbm.at[pl.ds(0, IDX_PER_CHUNK), :],
        rows_bufs[buf], sems[buf]).wait()

  def compute_mean(c, buf):
    rows = rows_bufs[buf]

    def mean_body(i, _):
      r = i * CTX_LEN
      accs = [rows[r, pl.ds(q * LANES, LANES)] for q in range(DV)]
      for j in range(1, CTX_LEN):
        for q in range(DV):
          accs[q] = accs[q] + rows[r + j, pl.ds(q * LANES, LANES)]
      b = c * CHUNK_A + i
      for q in range(DV):
        mean_v[b, pl.ds(q * LANES, LANES)] = accs[q] * (1.0 / CTX_LEN)
      return _

    lax.fori_loop(0, CHUNK_A, mean_body, 0, unroll=False)

  fire_gather(ctx_flat, 0, 0)

  def phase_a_pair(gp, _):
    c0 = 2 * gp
    fire_gather(ctx_flat, jnp.minimum(c0 + 1, N_CHUNK_A - 1), 1)
    wait_gather(0)
    compute_mean(c0, 0)
    fire_gather(ctx_flat, jnp.minimum(c0 + 2, N_CHUNK_A - 1), 0)
    wait_gather(1)
    compute_mean(c0 + 1, 1)
    return _

  lax.fori_loop(0, N_CHUNK_A // 2, phase_a_pair, 0, unroll=False)
  wait_gather(0)

  def compute_dots(c, buf):
    rows = rows_bufs[buf]

    def dot_body(i, _):
      b = c * CHUNK_B + i
      m = [mean_v[b, pl.ds(q * LANES, LANES)] for q in range(DV)]
      acc = jnp.zeros((LANES,), jnp.float32)
      for k in range(NUM_NEG):
        r = i * NUM_NEG + k
        p = rows[r, pl.ds(0, LANES)] * m[0]
        for q in range(1, DV):
          p = p + rows[r, pl.ds(q * LANES, LANES)] * m[q]
        for sidx in shuf_idx:
          p = p + _lane_shuffle(p, sidx)
        acc = jnp.where(k_masks[k], p, acc)
      plsc.store_scatter(out_v, [jnp.full((LANES,), i, jnp.int32), lane],
                         acc, mask=neg_mask)
      return _

    lax.fori_loop(0, CHUNK_B, dot_body, 0, unroll=False)
    pltpu.sync_copy(out_v,
                    out_hbm.at[pl.ds(b0 + c * CHUNK_B, CHUNK_B), :])

  fire_gather(neg_flat, 0, 0)

  def phase_b_pair(hp, _):
    c0 = 2 * hp
    fire_gather(neg_flat, jnp.minimum(c0 + 1, N_CHUNK_B - 1), 1)
    wait_gather(0)
    compute_dots(c0, 0)
    fire_gather(neg_flat, jnp.minimum(c0 + 2, N_CHUNK_B - 1), 0)
    wait_gather(1)
    compute_dots(c0 + 1, 1)
    return _

  lax.fori_loop(0, N_CHUNK_B // 2, phase_b_pair, 0, unroll=False)
  wait_gather(0)


@jax.jit
def _cbow_scores(ctx_idx, neg_idx, w_cat):
  mesh = plsc.VectorSubcoreMesh(
      core_axis_name="c", subcore_axis_name="s",
      num_cores=NC, num_subcores=NS)
  fn = pl.kernel(
      _sc_body,
      out_type=jax.ShapeDtypeStruct((BATCH, NUM_NEG), jnp.float32),
      mesh=mesh,
      compiler_params=pltpu.CompilerParams(use_tc_tiling_on_sc=False,
                                           needs_layout_passes=False),
      scratch_types=[
          pltpu.VMEM((B_PER_W, CTX_LEN), jnp.int32),
          pltpu.VMEM((B_PER_W, NUM_NEG), jnp.int32),
          pltpu.VMEM((B_PER_W * CTX_LEN,), jnp.int32),
          pltpu.VMEM((B_PER_W * NUM_NEG,), jnp.int32),
          pltpu.VMEM((IDX_PER_CHUNK, DIM), jnp.float32),
          pltpu.VMEM((IDX_PER_CHUNK, DIM), jnp.float32),
          pltpu.VMEM((B_PER_W, DIM), jnp.float32),
          pltpu.VMEM((CHUNK_B, NUM_NEG), jnp.float32),
          pltpu.SemaphoreType.DMA,
          pltpu.SemaphoreType.DMA,
      ],
  )
  return fn(ctx_idx, neg_idx, w_cat)


def kernel(context_words, negative_words, W_embed, W_context):
  w_cat = jnp.concatenate([W_embed, W_context], axis=1).reshape(
      2 * VOCAB, DIM)
  return _cbow_scores(context_words, negative_words, w_cat)
